# Initial kernel scaffold; baseline (speedup 1.0000x reference)
import jax
import jax.numpy as jnp
from jax.experimental import pallas as pl


def kernel(edge_index, embeds, qTrans, kTrans, vTrans):
    raise NotImplementedError("write your pallas kernel here")


# speedup vs baseline: 3.4539x; 3.4539x over previous
import functools

import jax
import jax.numpy as jnp
from jax import lax
from jax.experimental import pallas as pl
from jax.experimental.pallas import tpu as pltpu
from jax.experimental.pallas import tpu_sc as plsc

LAT = 128
HEAD = 4
HD = LAT // HEAD

NC = 2
NS = 16
NW = NC * NS

CH = 40


def _qkv_body(e_ref, qw_ref, kw_ref, vw_ref, q_ref, k_ref, v_ref):
    x = e_ref[...]
    q_ref[...] = jnp.dot(x, qw_ref[...], preferred_element_type=jnp.float32)
    k_ref[...] = jnp.dot(x, kw_ref[...], preferred_element_type=jnp.float32)
    v_ref[...] = jnp.dot(x, vw_ref[...], preferred_element_type=jnp.float32)


def _qkv(embeds, qw, kw, vw):
    n = embeds.shape[0]
    bn = 2000 if n % 2000 == 0 else n
    grid = (n // bn,)
    wspec = pl.BlockSpec((LAT, LAT), lambda i: (0, 0))
    nspec = pl.BlockSpec((bn, LAT), lambda i: (i, 0))
    out = jax.ShapeDtypeStruct((n, LAT), jnp.float32)
    return pl.pallas_call(
        _qkv_body,
        grid=grid,
        in_specs=[nspec, wspec, wspec, wspec],
        out_specs=[nspec, nspec, nspec],
        out_shape=[out, out, out],
    )(embeds, qw, kw, vw)


def _edge_body(n_nodes, n_edges, nau, rows_h, cols_h, an_h, q_h, k_h, v_h,
               u_out,
               ridx, cidx, sidx, qbuf, kbuf, vbuf, wstage, csb,
               u_sc,
               sem_q, sem_k, sem_v, sem_s):
    cid = lax.axis_index("c")
    sid = lax.axis_index("s")
    wid = cid * NS + sid
    epw = n_edges // NW
    nchunk = epw // CH

    zero16 = jnp.zeros((16,), jnp.float32)
    iota16 = lax.iota(jnp.int32, 16)
    mask4 = iota16 < 4

    def zw(i, c):
        for j in range(LAT // 16):
            wstage[i, pl.ds(j * 16, 16)] = zero16
        return c
    lax.fori_loop(0, 2 * CH, zw, 0)

    nrchunk = nau // (2 * CH)

    def zspan(kk, c):
        rck = sid + kk * NS

        @pl.when(rck < nrchunk)
        def _():
            pltpu.sync_copy(wstage, u_sc.at[pl.ds(rck * 2 * CH, 2 * CH)])
        return c
    lax.fori_loop(0, (nrchunk + NS - 1) // NS, zspan, 0)
    plsc.subcore_barrier()

    def chunk_body(ch, c):
        base = wid * epw + ch * CH
        pltpu.sync_copy(rows_h.at[pl.ds(base, CH)], ridx)
        pltpu.sync_copy(cols_h.at[pl.ds(base, CH)], cidx)
        pltpu.sync_copy(rows_h.at[pl.ds(base, CH)], sidx.at[pl.ds(0, CH)])
        pltpu.sync_copy(an_h.at[pl.ds(base, CH)], sidx.at[pl.ds(CH, CH)])
        cq = pltpu.async_copy(q_h.at[ridx], qbuf, sem_q)
        ck = pltpu.async_copy(k_h.at[cidx], kbuf, sem_k)
        cv = pltpu.async_copy(v_h.at[cidx], vbuf, sem_v)
        cq.wait()
        ck.wait()
        cv.wait()

        def edge_compute(e):
            prods = []
            for j in range(8):
                qv = qbuf[e, pl.ds(j * 16, 16)]
                kv = kbuf[e, pl.ds(j * 16, 16)]
                prods.append(qv * kv)
            ws = []
            for h in range(4):
                hs = prods[2 * h] + prods[2 * h + 1]
                tot = jnp.full((16,), jnp.sum(hs), jnp.float32)
                a = jnp.minimum(jnp.maximum(tot, -10.0), 10.0)
                ws.append(jnp.exp(a))
            wall = jnp.where(iota16 == 1, ws[1], ws[0])
            wall = jnp.where(iota16 == 2, ws[2], wall)
            wall = jnp.where(iota16 == 3, ws[3], wall)
            for j in range(8):
                vv = vbuf[e, pl.ds(j * 16, 16)]
                wstage[e, pl.ds(j * 16, 16)] = vv * ws[j // 2]
            rb = plsc.load_gather(ridx, [jnp.full((16,), e, jnp.int32)])
            rb7 = rb & 7
            wall4 = jnp.where(mask4, wall, 0.0)
            for j in range(8):
                blk = jnp.where(rb7 == j, wall4, 0.0)
                wstage[CH + e, pl.ds(j * 16, 16)] = blk

        def grp(g, c2):
            for l in range(16):
                edge_compute(g * 16 + l)
            return c2
        lax.fori_loop(0, CH // 16, grp, 0)
        for l in range(CH - (CH // 16) * 16):
            edge_compute((CH // 16) * 16 + l)

        pltpu.async_copy(wstage, u_sc.at[sidx], sem_s, add=True).wait()
        return c
    lax.fori_loop(0, nchunk, chunk_body, 0)

    plsc.subcore_barrier()

    def ospan(kk, c):
        rck = sid + kk * NS

        @pl.when(rck < nrchunk)
        def _():
            pltpu.sync_copy(u_sc.at[pl.ds(rck * 2 * CH, 2 * CH)],
                            u_out.at[cid, pl.ds(rck * 2 * CH, 2 * CH)])
        return c
    lax.fori_loop(0, (nrchunk + NS - 1) // NS, ospan, 0)


def _edge_pass(rows, cols, anrows, q, k, v):
    n = q.shape[0]
    e = rows.shape[0]
    nan_rows = (n // 8 + 2 * CH - 1) // (2 * CH) * (2 * CH)
    nau = n + nan_rows
    mesh = plsc.VectorSubcoreMesh(core_axis_name="c", subcore_axis_name="s")
    body = functools.partial(_edge_body, n, e, nau)
    fn = pl.kernel(
        body,
        out_type=[
            jax.ShapeDtypeStruct((NC, nau, LAT), jnp.float32),
        ],
        mesh=mesh,
        compiler_params=pltpu.CompilerParams(needs_layout_passes=False),
        scratch_types=[
            pltpu.VMEM((CH,), jnp.int32),
            pltpu.VMEM((CH,), jnp.int32),
            pltpu.VMEM((2 * CH,), jnp.int32),
            pltpu.VMEM((CH, LAT), jnp.float32),
            pltpu.VMEM((CH, LAT), jnp.float32),
            pltpu.VMEM((CH, LAT), jnp.float32),
            pltpu.VMEM((2 * CH, LAT), jnp.float32),
            pltpu.VMEM((64,), jnp.float32),
            pltpu.VMEM_SHARED((nau, LAT), jnp.float32),
            pltpu.SemaphoreType.DMA,
            pltpu.SemaphoreType.DMA,
            pltpu.SemaphoreType.DMA,
            pltpu.SemaphoreType.DMA,
        ],
    )
    return fn(rows, cols, anrows, q, k, v)


def _merge_body(u_ref, an_ref, o_ref):
    u = u_ref[0] + u_ref[1]
    an = an_ref[0] + an_ref[1]
    rr = lax.broadcasted_iota(jnp.int32, (16, LAT), 1) // HD
    hh = lax.broadcasted_iota(jnp.int32, (16, LAT), 0)
    expand = jnp.where(rr == hh, 1.0, 0.0).astype(jnp.float32)
    anx = jnp.dot(an, expand, preferred_element_type=jnp.float32)
    o_ref[...] = u / (anx + 1e-8)


def _merge(u_part, an16):
    n = u_part.shape[1]
    bn = 2000 if n % 2000 == 0 else n
    grid = (n // bn,)
    return pl.pallas_call(
        _merge_body,
        grid=grid,
        in_specs=[
            pl.BlockSpec((NC, bn, LAT), lambda i: (0, i, 0)),
            pl.BlockSpec((NC, bn, 16), lambda i: (0, i, 0)),
        ],
        out_specs=pl.BlockSpec((bn, LAT), lambda i: (i, 0)),
        out_shape=jax.ShapeDtypeStruct((n, LAT), jnp.float32),
    )(u_part, an16)


def kernel(edge_index, embeds, qTrans, kTrans, vTrans):
    rows = edge_index[0, :]
    cols = edge_index[1, :]
    n = embeds.shape[0]
    anrows = lax.shift_right_logical(rows, 3) + n
    q, k, v = _qkv(embeds, qTrans, kTrans, vTrans)
    (acc,) = _edge_pass(rows, cols, anrows, q, k, v)
    u_part = acc[:, :n, :]
    an16 = acc[:, n:, :].reshape(NC, -1, 16)[:, :n, :]
    return _merge(u_part, an16)
